# Initial kernel scaffold; baseline (speedup 1.0000x reference)
import jax
import jax.numpy as jnp
from jax.experimental import pallas as pl


def kernel(x, edge_index, edge_attr, global_attr, batch, Wx, bx, We, be, Wu, bu, Wedge, bedge, Wnode, bnode, Wglob, bglob):
    raise NotImplementedError("write your pallas kernel here")


# speedup vs baseline: 3.5654x; 3.5654x over previous
import functools

import jax
import jax.numpy as jnp
from jax import lax
from jax.experimental import pallas as pl
from jax.experimental.pallas import tpu as pltpu
from jax.experimental.pallas import tpu_sc as plsc

N = 10000
E = 320000
G = 16
H = 64
FN = 128
FE = 16
FG = 16

NC, NS = 2, 16
NW = NC * NS
EPW = E // NW
C = 200
K = EPW // C
NPW = N // NS
AW = 80

BN = 2000
BE = 2000

f32 = jnp.float32


def _edge_body(write_e, eW_hbm, ps_hbm, pd_hbm, src_hbm, dst_hbm, eg_hbm,
               *refs):
    if write_e:
        enew_hbm, nodeacc_hbm, graphacc_hbm = refs[0], refs[1], refs[2]
        scratch = refs[3:]
    else:
        enew_hbm = None
        nodeacc_hbm, graphacc_hbm = refs[0], refs[1]
        scratch = refs[2:]
    (src_v, dst_v, eg_v, Abuf, Bbuf, Ebuf, Sbuf, Obuf, node_sh, graph_sh,
     sem_a, sem_b, sem_e, sem_o, sem_n, sem_g) = scratch

    c = lax.axis_index("c")
    s = lax.axis_index("s")
    wid = c * NS + s

    zero16 = jnp.zeros((16,), f32)

    @pl.loop(0, C)
    def _(i):
        for j in range(AW // 16):
            Sbuf[i, pl.ds(j * 16, 16)] = zero16

    @pl.loop(s, N // C, step=NS)
    def _(j):
        pltpu.sync_copy(Sbuf, node_sh.at[pl.ds(j * C, C)])

    @pl.when(s == 0)
    def _():
        pltpu.sync_copy(Sbuf.at[pl.ds(0, G)], graph_sh)

    cpat = jnp.where(lax.iota(jnp.int32, 16) == 0, 1.0, 0.0).astype(f32)

    @pl.loop(0, C)
    def _(i):
        Sbuf[i, pl.ds(H, 16)] = cpat

    plsc.subcore_barrier()

    @pl.loop(0, K)
    def _(k):
        base = wid * EPW + k * C
        pltpu.sync_copy(src_hbm.at[pl.ds(base, C)], src_v)
        pltpu.sync_copy(dst_hbm.at[pl.ds(base, C)], dst_v)
        pltpu.sync_copy(eg_hbm.at[pl.ds(base, C)], eg_v)
        ca = pltpu.async_copy(ps_hbm.at[src_v], Abuf, sem_a)
        cb = pltpu.async_copy(pd_hbm.at[dst_v], Bbuf, sem_b)
        ce = pltpu.async_copy(eW_hbm.at[pl.ds(base, C)], Ebuf, sem_e)
        ca.wait()
        cb.wait()
        ce.wait()

        @pl.loop(0, C)
        def _(i):
            for j in range(H // 16):
                sl = pl.ds(j * 16, 16)
                v = jnp.maximum(Ebuf[i, sl] + Abuf[i, sl] + Bbuf[i, sl], 0.0)
                Sbuf[i, sl] = v
                if write_e:
                    Obuf[i, sl] = v

        if write_e:
            co = pltpu.async_copy(Obuf, enew_hbm.at[pl.ds(base, C)], sem_o)
        cn = pltpu.async_copy(Sbuf, node_sh.at[dst_v], sem_n, add=True)
        cg = pltpu.async_copy(Sbuf, graph_sh.at[eg_v], sem_g, add=True)
        cn.wait()
        cg.wait()
        if write_e:
            co.wait()

    plsc.subcore_barrier()

    @pl.loop(s, N // C, step=NS)
    def _(j):
        pltpu.sync_copy(node_sh.at[pl.ds(j * C, C)],
                        nodeacc_hbm.at[c].at[pl.ds(j * C, C)])

    @pl.when(s == 0)
    def _():
        pltpu.sync_copy(graph_sh, graphacc_hbm.at[c])


def _make_edge_kernel(write_e):
    outs = []
    if write_e:
        outs.append(jax.ShapeDtypeStruct((E, H), f32))
    outs.append(jax.ShapeDtypeStruct((NC, N, AW), f32))
    outs.append(jax.ShapeDtypeStruct((NC, G, AW), f32))
    return pl.kernel(
        functools.partial(_edge_body, write_e),
        out_type=tuple(outs),
        compiler_params=pltpu.CompilerParams(use_tc_tiling_on_sc=False),
        mesh=plsc.VectorSubcoreMesh(core_axis_name="c", subcore_axis_name="s"),
        scratch_types=[
            pltpu.VMEM((C,), jnp.int32),
            pltpu.VMEM((C,), jnp.int32),
            pltpu.VMEM((C,), jnp.int32),
            pltpu.VMEM((C, H), f32),
            pltpu.VMEM((C, H), f32),
            pltpu.VMEM((C, H), f32),
            pltpu.VMEM((C, AW), f32),
            pltpu.VMEM((C, H), f32),
            pltpu.VMEM_SHARED((N, AW), f32),
            pltpu.VMEM_SHARED((G, AW), f32),
            pltpu.SemaphoreType.DMA,
            pltpu.SemaphoreType.DMA,
            pltpu.SemaphoreType.DMA,
            pltpu.SemaphoreType.DMA,
            pltpu.SemaphoreType.DMA,
            pltpu.SemaphoreType.DMA,
        ],
    )


_edge_kernel_we = _make_edge_kernel(True)
_edge_kernel_last = _make_edge_kernel(False)


def _u1_body(x_ref, Wx_ref, bx_ref, Ws_ref, Wd_ref, h_ref, ps_ref, pd_ref):
    h = jax.nn.relu(jnp.dot(x_ref[...], Wx_ref[...],
                            preferred_element_type=f32) + bx_ref[...])
    h_ref[...] = h
    ps_ref[...] = jnp.dot(h, Ws_ref[...], preferred_element_type=f32)
    pd_ref[...] = jnp.dot(h, Wd_ref[...], preferred_element_type=f32)


_u1_kernel = pl.pallas_call(
    _u1_body,
    grid=(N // BN,),
    in_specs=[
        pl.BlockSpec((BN, FN), lambda m: (m, 0)),
        pl.BlockSpec((FN, H), lambda m: (0, 0)),
        pl.BlockSpec((1, H), lambda m: (0, 0)),
        pl.BlockSpec((H, H), lambda m: (0, 0)),
        pl.BlockSpec((H, H), lambda m: (0, 0)),
    ],
    out_specs=[
        pl.BlockSpec((BN, H), lambda m: (m, 0)),
        pl.BlockSpec((BN, H), lambda m: (m, 0)),
        pl.BlockSpec((BN, H), lambda m: (m, 0)),
    ],
    out_shape=[
        jax.ShapeDtypeStruct((N, H), f32),
        jax.ShapeDtypeStruct((N, H), f32),
        jax.ShapeDtypeStruct((N, H), f32),
    ],
)


def _u2enc_body(ea_ref, src_ref, off_lo_ref, off_hi_ref, We_ref, be_ref,
                Wecol_ref, gvec_ref, out_ref, eg_ref):
    e0 = jax.nn.relu(jnp.dot(ea_ref[...], We_ref[...],
                             preferred_element_type=f32) + be_ref[...])
    srcf = src_ref[...].astype(f32)
    oh = ((srcf >= off_lo_ref[...]).astype(f32)
          - (srcf >= off_hi_ref[...]).astype(f32))
    out_ref[...] = (jnp.dot(e0, Wecol_ref[...], preferred_element_type=f32)
                    + jnp.dot(oh, gvec_ref[...], preferred_element_type=f32))
    eg_ref[...] = jnp.dot(
        oh, lax.broadcasted_iota(jnp.int32, (G, 1), 0).astype(f32),
        preferred_element_type=f32).astype(jnp.int32)


_u2enc_kernel = pl.pallas_call(
    _u2enc_body,
    grid=(E // BE,),
    in_specs=[
        pl.BlockSpec((BE, FE), lambda m: (m, 0)),
        pl.BlockSpec((BE, 1), lambda m: (m, 0)),
        pl.BlockSpec((1, G), lambda m: (0, 0)),
        pl.BlockSpec((1, G), lambda m: (0, 0)),
        pl.BlockSpec((FE, H), lambda m: (0, 0)),
        pl.BlockSpec((1, H), lambda m: (0, 0)),
        pl.BlockSpec((H, H), lambda m: (0, 0)),
        pl.BlockSpec((G, H), lambda m: (0, 0)),
    ],
    out_specs=[
        pl.BlockSpec((BE, H), lambda m: (m, 0)),
        pl.BlockSpec((BE, 1), lambda m: (m, 0)),
    ],
    out_shape=[
        jax.ShapeDtypeStruct((E, H), f32),
        jax.ShapeDtypeStruct((E, 1), jnp.int32),
    ],
)


def _u2mid_body(e_ref, eg_ref, Wecol_ref, gvec_ref, out_ref):
    oh = (eg_ref[...] == lax.broadcasted_iota(jnp.int32, (BE, G), 1)).astype(f32)
    out_ref[...] = (jnp.dot(e_ref[...], Wecol_ref[...], preferred_element_type=f32)
                    + jnp.dot(oh, gvec_ref[...], preferred_element_type=f32))


_u2mid_kernel = pl.pallas_call(
    _u2mid_body,
    grid=(E // BE,),
    in_specs=[
        pl.BlockSpec((BE, H), lambda m: (m, 0)),
        pl.BlockSpec((BE, 1), lambda m: (m, 0)),
        pl.BlockSpec((H, H), lambda m: (0, 0)),
        pl.BlockSpec((G, H), lambda m: (0, 0)),
    ],
    out_specs=[pl.BlockSpec((BE, H), lambda m: (m, 0))],
    out_shape=[jax.ShapeDtypeStruct((E, H), f32)],
)


def _u4_body(with_proj, acc0_ref, acc1_ref, h_ref, b_ref, gn_ref, Wn2_ref,
             *refs):
    if with_proj:
        Ws_ref, Wd_ref, h_out, ps_ref, pd_ref, nsum_ref, cnt_ref = refs
    else:
        h_out, nsum_ref, cnt_ref = refs
    m = pl.program_id(0)
    acc = acc0_ref[...] + acc1_ref[...]
    agg = acc[:, :H] / jnp.clip(acc[:, H:H + 1], 1.0, None)
    W2 = Wn2_ref[...]
    oh = (b_ref[...] == lax.broadcasted_iota(jnp.int32, (BN, G), 1)).astype(f32)
    hn = jax.nn.relu(jnp.dot(h_ref[...], W2[:H], preferred_element_type=f32)
                     + jnp.dot(agg, W2[H:], preferred_element_type=f32)
                     + jnp.dot(oh, gn_ref[...], preferred_element_type=f32))
    h_out[...] = hn
    if with_proj:
        ps_ref[...] = jnp.dot(hn, Ws_ref[...], preferred_element_type=f32)
        pd_ref[...] = jnp.dot(hn, Wd_ref[...], preferred_element_type=f32)

    @pl.when(m == 0)
    def _():
        nsum_ref[...] = jnp.zeros_like(nsum_ref)
        cnt_ref[...] = jnp.zeros_like(cnt_ref)

    nsum_ref[...] += lax.dot_general(oh, hn, (((0,), (0,)), ((), ())),
                                     preferred_element_type=f32)
    cnt_ref[...] += lax.dot_general(oh, jnp.ones((BN, 1), f32),
                                    (((0,), (0,)), ((), ())),
                                    preferred_element_type=f32)


def _make_u4(with_proj):
    in_specs = [
        pl.BlockSpec((BN, AW), lambda m: (m, 0)),
        pl.BlockSpec((BN, AW), lambda m: (m, 0)),
        pl.BlockSpec((BN, H), lambda m: (m, 0)),
        pl.BlockSpec((BN, 1), lambda m: (m, 0)),
        pl.BlockSpec((G, H), lambda m: (0, 0)),
        pl.BlockSpec((2 * H, H), lambda m: (0, 0)),
    ]
    out_specs = [pl.BlockSpec((BN, H), lambda m: (m, 0))]
    out_shape = [jax.ShapeDtypeStruct((N, H), f32)]
    if with_proj:
        in_specs += [
            pl.BlockSpec((H, H), lambda m: (0, 0)),
            pl.BlockSpec((H, H), lambda m: (0, 0)),
        ]
        out_specs += [
            pl.BlockSpec((BN, H), lambda m: (m, 0)),
            pl.BlockSpec((BN, H), lambda m: (m, 0)),
        ]
        out_shape += [
            jax.ShapeDtypeStruct((N, H), f32),
            jax.ShapeDtypeStruct((N, H), f32),
        ]
    out_specs += [
        pl.BlockSpec((G, H), lambda m: (0, 0)),
        pl.BlockSpec((G, 1), lambda m: (0, 0)),
    ]
    out_shape += [
        jax.ShapeDtypeStruct((G, H), f32),
        jax.ShapeDtypeStruct((G, 1), f32),
    ]
    return pl.pallas_call(
        functools.partial(_u4_body, with_proj),
        grid=(N // BN,),
        in_specs=in_specs,
        out_specs=out_specs,
        out_shape=out_shape,
    )


_u4_kernel = _make_u4(True)
_u4_last_kernel = _make_u4(False)


def _gkenc_body(ga_ref, batch_ref, Wu_ref, bu_ref, Wg_ref, bedge_ref, Wng_ref,
                bnode_ref, g_ref, gvec_ref, gn_ref, off_lo_ref, off_hi_ref):
    g = jax.nn.relu(jnp.dot(ga_ref[...], Wu_ref[...],
                            preferred_element_type=f32) + bu_ref[...])
    g_ref[...] = g
    gvec_ref[...] = jnp.dot(g, Wg_ref[...], preferred_element_type=f32) + bedge_ref[...]
    gn_ref[...] = jnp.dot(g, Wng_ref[...], preferred_element_type=f32) + bnode_ref[...]
    b = batch_ref[...]
    it = lax.broadcasted_iota(jnp.int32, (N, G), 1)
    off_lo_ref[...] = jnp.sum((b < it).astype(f32), axis=0, keepdims=True)
    off_hi_ref[...] = jnp.sum((b <= it).astype(f32), axis=0, keepdims=True)


_gkenc_kernel = pl.pallas_call(
    _gkenc_body,
    out_shape=[
        jax.ShapeDtypeStruct((G, H), f32),
        jax.ShapeDtypeStruct((G, H), f32),
        jax.ShapeDtypeStruct((G, H), f32),
        jax.ShapeDtypeStruct((1, G), f32),
        jax.ShapeDtypeStruct((1, G), f32),
    ],
)


def _global_update(g, nsum, cnt, gacc0, gacc1, Wg2, bglob):
    node_mean = nsum / jnp.clip(cnt, 1.0, None)
    gacc = gacc0 + gacc1
    edge_mean = gacc[:, :H] / jnp.clip(gacc[:, H:H + 1], 1.0, None)
    return jax.nn.relu(jnp.dot(g, Wg2[:H], preferred_element_type=f32)
                       + jnp.dot(node_mean, Wg2[H:2 * H], preferred_element_type=f32)
                       + jnp.dot(edge_mean, Wg2[2 * H:], preferred_element_type=f32)
                       + bglob)


def _gkup_body(g_ref, nsum_ref, cnt_ref, gacc0_ref, gacc1_ref, Wg2_ref,
               bglob_ref, Wg_ref, bedge_ref, Wng_ref, bnode_ref,
               gout_ref, gvec_ref, gn_ref):
    gnew = _global_update(g_ref[...], nsum_ref[...], cnt_ref[...],
                          gacc0_ref[...], gacc1_ref[...], Wg2_ref[...],
                          bglob_ref[...])
    gout_ref[...] = gnew
    gvec_ref[...] = jnp.dot(gnew, Wg_ref[...], preferred_element_type=f32) + bedge_ref[...]
    gn_ref[...] = jnp.dot(gnew, Wng_ref[...], preferred_element_type=f32) + bnode_ref[...]


_gkup_kernel = pl.pallas_call(
    _gkup_body,
    out_shape=[
        jax.ShapeDtypeStruct((G, H), f32),
        jax.ShapeDtypeStruct((G, H), f32),
        jax.ShapeDtypeStruct((G, H), f32),
    ],
)


def _gkfin_body(g_ref, nsum_ref, cnt_ref, gacc0_ref, gacc1_ref, Wg2_ref,
                bglob_ref, out_ref):
    gnew = _global_update(g_ref[...], nsum_ref[...], cnt_ref[...],
                          gacc0_ref[...], gacc1_ref[...], Wg2_ref[...],
                          bglob_ref[...])
    avg = nsum_ref[...] / jnp.clip(cnt_ref[...], 1.0, None)
    out_ref[...] = (jnp.sum(gnew, axis=1, keepdims=True)
                    + jnp.sum(avg, axis=1, keepdims=True))


_gkfin_kernel = pl.pallas_call(
    _gkfin_body,
    out_shape=[jax.ShapeDtypeStruct((G, 1), f32)],
)


def kernel(x, edge_index, edge_attr, global_attr, batch,
           Wx, bx, We, be, Wu, bu, Wedge, bedge, Wnode, bnode, Wglob, bglob):
    x = x.reshape(N, FN)
    edge_attr = edge_attr.reshape(E, FE)
    global_attr = global_attr.reshape(G, FG)
    src = edge_index[0].astype(jnp.int32)
    dst = edge_index[1].astype(jnp.int32)
    batch_i = batch.astype(jnp.int32)

    Ws = Wedge[:, 0:H, :]
    Wd = Wedge[:, H:2 * H, :]
    Wecol = Wedge[:, 2 * H:3 * H, :]
    Wgcol = Wedge[:, 3 * H:, :]
    Wn2 = Wnode[:, 0:2 * H, :]
    Wng = Wnode[:, 2 * H:, :]
    bx2 = bx.reshape(1, H)
    be2 = be.reshape(1, H)
    bu2 = bu.reshape(1, H)
    bedge2 = bedge.reshape(3, 1, H)
    bnode2 = bnode.reshape(3, 1, H)
    bglob2 = bglob.reshape(3, 1, H)

    b2 = batch_i.reshape(N, 1)
    src2 = src.reshape(E, 1)

    g, gvec, gn, off_lo, off_hi = _gkenc_kernel(
        global_attr, b2, Wu, bu2, Wgcol[0], bedge2[0], Wng[0], bnode2[0])
    h, ps, pd = _u1_kernel(x, Wx, bx2, Ws[0], Wd[0])
    eW, eg2 = _u2enc_kernel(edge_attr, src2, off_lo, off_hi, We, be2,
                            Wecol[0], gvec)
    eg = eg2.reshape(E)

    e_new = None
    for t in range(3):
        if t > 0:
            eW = _u2mid_kernel(e_new, eg2, Wecol[t], gvec)[0]
        if t < 2:
            e_new, nacc, gacc = _edge_kernel_we(eW, ps, pd, src, dst, eg)
            h, ps, pd, nsum, cnt = _u4_kernel(
                nacc[0], nacc[1], h, b2, gn, Wn2[t], Ws[t + 1], Wd[t + 1])
            g, gvec, gn = _gkup_kernel(
                g, nsum, cnt, gacc[0], gacc[1], Wglob[t], bglob2[t],
                Wgcol[t + 1], bedge2[t + 1], Wng[t + 1], bnode2[t + 1])
        else:
            nacc, gacc = _edge_kernel_last(eW, ps, pd, src, dst, eg)
            h, nsum, cnt = _u4_last_kernel(
                nacc[0], nacc[1], h, b2, gn, Wn2[t])
            out = _gkfin_kernel(g, nsum, cnt, gacc[0], gacc[1], Wglob[t],
                                bglob2[t])[0]

    return out.reshape(G)


# speedup vs baseline: 3.8878x; 1.0904x over previous
import functools

import jax
import jax.numpy as jnp
from jax import lax
from jax.experimental import pallas as pl
from jax.experimental.pallas import tpu as pltpu
from jax.experimental.pallas import tpu_sc as plsc

N = 10000
E = 320000
G = 16
H = 64
FN = 128
FE = 16
FG = 16

NC, NS = 2, 16
NW = NC * NS
EPW = E // NW
C = 200
K = EPW // C
AW = 80

BN = 2000
BE = 2000

f32 = jnp.float32


def _edge_body(write_e, eW_hbm, ps_hbm, pd_hbm, src_hbm, dst_hbm, eg_hbm,
               *refs):
    if write_e:
        enew_hbm, nodeacc_hbm, graphacc_hbm = refs[0], refs[1], refs[2]
        scratch = refs[3:]
    else:
        enew_hbm = None
        nodeacc_hbm, graphacc_hbm = refs[0], refs[1]
        scratch = refs[2:]
    (src_v, dst_v, eg_v, Abuf, Bbuf, Ebuf, Sbuf, Obuf, node_sh, graph_sh,
     sem_a, sem_b, sem_e, sem_o, sem_n, sem_g) = scratch

    c = lax.axis_index("c")
    s = lax.axis_index("s")
    wid = c * NS + s

    zero16 = jnp.zeros((16,), f32)

    @pl.loop(0, C)
    def _(i):
        for j in range(AW // 16):
            Sbuf[i, pl.ds(j * 16, 16)] = zero16

    @pl.loop(s, N // C, step=NS)
    def _(j):
        pltpu.sync_copy(Sbuf, node_sh.at[pl.ds(j * C, C)])

    @pl.when(s == 0)
    def _():
        pltpu.sync_copy(Sbuf.at[pl.ds(0, G)], graph_sh)

    cpat = jnp.where(lax.iota(jnp.int32, 16) == 0, 1.0, 0.0).astype(f32)

    @pl.loop(0, C)
    def _(i):
        Sbuf[i, pl.ds(H, 16)] = cpat

    plsc.subcore_barrier()

    @pl.loop(0, K)
    def _(k):
        base = wid * EPW + k * C
        pltpu.sync_copy(src_hbm.at[pl.ds(base, C)], src_v)
        pltpu.sync_copy(dst_hbm.at[pl.ds(base, C)], dst_v)
        pltpu.sync_copy(eg_hbm.at[pl.ds(base, C)], eg_v)
        ca = pltpu.async_copy(ps_hbm.at[src_v], Abuf, sem_a)
        cb = pltpu.async_copy(pd_hbm.at[dst_v], Bbuf, sem_b)
        ce = pltpu.async_copy(eW_hbm.at[pl.ds(base, C)], Ebuf, sem_e)
        ca.wait()
        cb.wait()
        ce.wait()

        @pl.loop(0, C)
        def _(i):
            for j in range(H // 16):
                sl = pl.ds(j * 16, 16)
                v = jnp.maximum(Ebuf[i, sl] + Abuf[i, sl] + Bbuf[i, sl], 0.0)
                Sbuf[i, sl] = v
                if write_e:
                    Obuf[i, sl] = v

        if write_e:
            co = pltpu.async_copy(Obuf, enew_hbm.at[pl.ds(base, C)], sem_o)
        cn = pltpu.async_copy(Sbuf, node_sh.at[dst_v], sem_n, add=True)
        cg = pltpu.async_copy(Sbuf, graph_sh.at[eg_v], sem_g, add=True)
        cn.wait()
        cg.wait()
        if write_e:
            co.wait()

    plsc.subcore_barrier()

    @pl.loop(s, N // C, step=NS)
    def _(j):
        pltpu.sync_copy(node_sh.at[pl.ds(j * C, C)],
                        nodeacc_hbm.at[c].at[pl.ds(j * C, C)])

    @pl.when(s == 0)
    def _():
        pltpu.sync_copy(graph_sh, graphacc_hbm.at[c])


def _make_edge_kernel(write_e):
    outs = []
    if write_e:
        outs.append(jax.ShapeDtypeStruct((E, H), f32))
    outs.append(jax.ShapeDtypeStruct((NC, N, AW), f32))
    outs.append(jax.ShapeDtypeStruct((NC, G, AW), f32))
    return pl.kernel(
        functools.partial(_edge_body, write_e),
        out_type=tuple(outs),
        compiler_params=pltpu.CompilerParams(use_tc_tiling_on_sc=False),
        mesh=plsc.VectorSubcoreMesh(core_axis_name="c", subcore_axis_name="s"),
        scratch_types=[
            pltpu.VMEM((C,), jnp.int32),
            pltpu.VMEM((C,), jnp.int32),
            pltpu.VMEM((C,), jnp.int32),
            pltpu.VMEM((C, H), f32),
            pltpu.VMEM((C, H), f32),
            pltpu.VMEM((C, H), f32),
            pltpu.VMEM((C, AW), f32),
            pltpu.VMEM((C, H), f32),
            pltpu.VMEM_SHARED((N, AW), f32),
            pltpu.VMEM_SHARED((G, AW), f32),
            pltpu.SemaphoreType.DMA,
            pltpu.SemaphoreType.DMA,
            pltpu.SemaphoreType.DMA,
            pltpu.SemaphoreType.DMA,
            pltpu.SemaphoreType.DMA,
            pltpu.SemaphoreType.DMA,
        ],
    )


_edge_kernel_we = _make_edge_kernel(True)
_edge_kernel_last = _make_edge_kernel(False)


def _rowwise_onehot(m, off_lo, off_hi):
    rid = (lax.broadcasted_iota(jnp.int32, (BN, 1), 0).astype(f32)
           + (m * BN).astype(f32))
    return (rid >= off_lo).astype(f32) - (rid >= off_hi).astype(f32)


def _u1_body(x_ref, Wx_ref, bx_ref, Ws_ref, Wd_ref, gvec_ref, off_lo_ref,
             off_hi_ref, h_ref, ps_ref, pd_ref):
    m = pl.program_id(0)
    h = jax.nn.relu(jnp.dot(x_ref[...], Wx_ref[...],
                            preferred_element_type=f32) + bx_ref[...])
    h_ref[...] = h
    oh = _rowwise_onehot(m, off_lo_ref[...], off_hi_ref[...])
    ps_ref[...] = (jnp.dot(h, Ws_ref[...], preferred_element_type=f32)
                   + jnp.dot(oh, gvec_ref[...], preferred_element_type=f32))
    pd_ref[...] = jnp.dot(h, Wd_ref[...], preferred_element_type=f32)


_u1_kernel = pl.pallas_call(
    _u1_body,
    grid=(N // BN,),
    in_specs=[
        pl.BlockSpec((BN, FN), lambda m: (m, 0)),
        pl.BlockSpec((FN, H), lambda m: (0, 0)),
        pl.BlockSpec((1, H), lambda m: (0, 0)),
        pl.BlockSpec((H, H), lambda m: (0, 0)),
        pl.BlockSpec((H, H), lambda m: (0, 0)),
        pl.BlockSpec((G, H), lambda m: (0, 0)),
        pl.BlockSpec((1, G), lambda m: (0, 0)),
        pl.BlockSpec((1, G), lambda m: (0, 0)),
    ],
    out_specs=[
        pl.BlockSpec((BN, H), lambda m: (m, 0)),
        pl.BlockSpec((BN, H), lambda m: (m, 0)),
        pl.BlockSpec((BN, H), lambda m: (m, 0)),
    ],
    out_shape=[
        jax.ShapeDtypeStruct((N, H), f32),
        jax.ShapeDtypeStruct((N, H), f32),
        jax.ShapeDtypeStruct((N, H), f32),
    ],
)


def _uP_body(h_ref, Ws_ref, Wd_ref, gvec_ref, off_lo_ref, off_hi_ref,
             ps_ref, pd_ref):
    m = pl.program_id(0)
    h = h_ref[...]
    oh = _rowwise_onehot(m, off_lo_ref[...], off_hi_ref[...])
    ps_ref[...] = (jnp.dot(h, Ws_ref[...], preferred_element_type=f32)
                   + jnp.dot(oh, gvec_ref[...], preferred_element_type=f32))
    pd_ref[...] = jnp.dot(h, Wd_ref[...], preferred_element_type=f32)


_uP_kernel = pl.pallas_call(
    _uP_body,
    grid=(N // BN,),
    in_specs=[
        pl.BlockSpec((BN, H), lambda m: (m, 0)),
        pl.BlockSpec((H, H), lambda m: (0, 0)),
        pl.BlockSpec((H, H), lambda m: (0, 0)),
        pl.BlockSpec((G, H), lambda m: (0, 0)),
        pl.BlockSpec((1, G), lambda m: (0, 0)),
        pl.BlockSpec((1, G), lambda m: (0, 0)),
    ],
    out_specs=[
        pl.BlockSpec((BN, H), lambda m: (m, 0)),
        pl.BlockSpec((BN, H), lambda m: (m, 0)),
    ],
    out_shape=[
        jax.ShapeDtypeStruct((N, H), f32),
        jax.ShapeDtypeStruct((N, H), f32),
    ],
)


def _u2enc_body(ea_ref, We_ref, be_ref, Wecol_ref, out_ref):
    e0 = jax.nn.relu(jnp.dot(ea_ref[...], We_ref[...],
                             preferred_element_type=f32) + be_ref[...])
    out_ref[...] = jnp.dot(e0, Wecol_ref[...], preferred_element_type=f32)


_u2enc_kernel = pl.pallas_call(
    _u2enc_body,
    grid=(E // BE,),
    in_specs=[
        pl.BlockSpec((BE, FE), lambda m: (m, 0)),
        pl.BlockSpec((FE, H), lambda m: (0, 0)),
        pl.BlockSpec((1, H), lambda m: (0, 0)),
        pl.BlockSpec((H, H), lambda m: (0, 0)),
    ],
    out_specs=[pl.BlockSpec((BE, H), lambda m: (m, 0))],
    out_shape=[jax.ShapeDtypeStruct((E, H), f32)],
)


def _u2mid_body(e_ref, Wecol_ref, out_ref):
    out_ref[...] = jnp.dot(e_ref[...], Wecol_ref[...],
                           preferred_element_type=f32)


_u2mid_kernel = pl.pallas_call(
    _u2mid_body,
    grid=(E // BE,),
    in_specs=[
        pl.BlockSpec((BE, H), lambda m: (m, 0)),
        pl.BlockSpec((H, H), lambda m: (0, 0)),
    ],
    out_specs=[pl.BlockSpec((BE, H), lambda m: (m, 0))],
    out_shape=[jax.ShapeDtypeStruct((E, H), f32)],
)


def _eg_body(src_hbm, off_hbm, eg_hbm, src_v, off_v, eg_v):
    c = lax.axis_index("c")
    s = lax.axis_index("s")
    wid = c * NS + s
    base = wid * EPW
    pltpu.sync_copy(src_hbm.at[pl.ds(base, EPW)], src_v)
    pltpu.sync_copy(off_hbm, off_v)
    off_vec = off_v[pl.ds(0, 16)]
    offs = [off_vec[j] for j in range(G)]

    @pl.loop(0, EPW // 16)
    def _(i):
        v = src_v[pl.ds(i * 16, 16)]
        acc = jnp.zeros((16,), jnp.int32)
        for j in range(G):
            acc = acc + jnp.where(v >= offs[j], 1, 0).astype(jnp.int32)
        eg_v[pl.ds(i * 16, 16)] = acc

    pltpu.sync_copy(eg_v, eg_hbm.at[pl.ds(base, EPW)])


_eg_kernel = pl.kernel(
    _eg_body,
    out_type=jax.ShapeDtypeStruct((E,), jnp.int32),
    compiler_params=pltpu.CompilerParams(use_tc_tiling_on_sc=False),
    mesh=plsc.VectorSubcoreMesh(core_axis_name="c", subcore_axis_name="s"),
    scratch_types=[
        pltpu.VMEM((EPW,), jnp.int32),
        pltpu.VMEM((16,), jnp.int32),
        pltpu.VMEM((EPW,), jnp.int32),
    ],
)


def _u4_body(acc0_ref, acc1_ref, h_ref, gn_ref, Wn2_ref, off_lo_ref,
             off_hi_ref, h_out, nsum_ref, cnt_ref):
    m = pl.program_id(0)
    acc = acc0_ref[...] + acc1_ref[...]
    agg = acc[:, :H] / jnp.clip(acc[:, H:H + 1], 1.0, None)
    W2 = Wn2_ref[...]
    oh = _rowwise_onehot(m, off_lo_ref[...], off_hi_ref[...])
    hn = jax.nn.relu(jnp.dot(h_ref[...], W2[:H], preferred_element_type=f32)
                     + jnp.dot(agg, W2[H:], preferred_element_type=f32)
                     + jnp.dot(oh, gn_ref[...], preferred_element_type=f32))
    h_out[...] = hn

    @pl.when(m == 0)
    def _():
        nsum_ref[...] = jnp.zeros_like(nsum_ref)
        cnt_ref[...] = jnp.zeros_like(cnt_ref)

    nsum_ref[...] += lax.dot_general(oh, hn, (((0,), (0,)), ((), ())),
                                     preferred_element_type=f32)
    cnt_ref[...] += lax.dot_general(oh, jnp.ones((BN, 1), f32),
                                    (((0,), (0,)), ((), ())),
                                    preferred_element_type=f32)


_u4_kernel = pl.pallas_call(
    _u4_body,
    grid=(N // BN,),
    in_specs=[
        pl.BlockSpec((BN, AW), lambda m: (m, 0)),
        pl.BlockSpec((BN, AW), lambda m: (m, 0)),
        pl.BlockSpec((BN, H), lambda m: (m, 0)),
        pl.BlockSpec((G, H), lambda m: (0, 0)),
        pl.BlockSpec((2 * H, H), lambda m: (0, 0)),
        pl.BlockSpec((1, G), lambda m: (0, 0)),
        pl.BlockSpec((1, G), lambda m: (0, 0)),
    ],
    out_specs=[
        pl.BlockSpec((BN, H), lambda m: (m, 0)),
        pl.BlockSpec((G, H), lambda m: (0, 0)),
        pl.BlockSpec((G, 1), lambda m: (0, 0)),
    ],
    out_shape=[
        jax.ShapeDtypeStruct((N, H), f32),
        jax.ShapeDtypeStruct((G, H), f32),
        jax.ShapeDtypeStruct((G, 1), f32),
    ],
)


def _gkenc_body(ga_ref, batch_ref, Wu_ref, bu_ref, Wg_ref, bedge_ref, Wng_ref,
                bnode_ref, g_ref, gvec_ref, gn_ref, off_lo_ref, off_hi_ref,
                off_i_ref):
    g = jax.nn.relu(jnp.dot(ga_ref[...], Wu_ref[...],
                            preferred_element_type=f32) + bu_ref[...])
    g_ref[...] = g
    gvec_ref[...] = jnp.dot(g, Wg_ref[...], preferred_element_type=f32) + bedge_ref[...]
    gn_ref[...] = jnp.dot(g, Wng_ref[...], preferred_element_type=f32) + bnode_ref[...]
    b = batch_ref[...]
    it = lax.broadcasted_iota(jnp.int32, (N, G), 1)
    off_lo_ref[...] = jnp.sum((b < it).astype(f32), axis=0, keepdims=True)
    hi = jnp.sum((b <= it).astype(f32), axis=0, keepdims=True)
    off_hi_ref[...] = hi
    off_i_ref[...] = hi.astype(jnp.int32)


_gkenc_kernel = pl.pallas_call(
    _gkenc_body,
    out_shape=[
        jax.ShapeDtypeStruct((G, H), f32),
        jax.ShapeDtypeStruct((G, H), f32),
        jax.ShapeDtypeStruct((G, H), f32),
        jax.ShapeDtypeStruct((1, G), f32),
        jax.ShapeDtypeStruct((1, G), f32),
        jax.ShapeDtypeStruct((1, G), jnp.int32),
    ],
)


def _global_update(g, nsum, cnt, gacc0, gacc1, Wg2, bglob):
    node_mean = nsum / jnp.clip(cnt, 1.0, None)
    gacc = gacc0 + gacc1
    edge_mean = gacc[:, :H] / jnp.clip(gacc[:, H:H + 1], 1.0, None)
    return jax.nn.relu(jnp.dot(g, Wg2[:H], preferred_element_type=f32)
                       + jnp.dot(node_mean, Wg2[H:2 * H], preferred_element_type=f32)
                       + jnp.dot(edge_mean, Wg2[2 * H:], preferred_element_type=f32)
                       + bglob)


def _gkup_body(g_ref, nsum_ref, cnt_ref, gacc0_ref, gacc1_ref, Wg2_ref,
               bglob_ref, Wg_ref, bedge_ref, Wng_ref, bnode_ref,
               gout_ref, gvec_ref, gn_ref):
    gnew = _global_update(g_ref[...], nsum_ref[...], cnt_ref[...],
                          gacc0_ref[...], gacc1_ref[...], Wg2_ref[...],
                          bglob_ref[...])
    gout_ref[...] = gnew
    gvec_ref[...] = jnp.dot(gnew, Wg_ref[...], preferred_element_type=f32) + bedge_ref[...]
    gn_ref[...] = jnp.dot(gnew, Wng_ref[...], preferred_element_type=f32) + bnode_ref[...]


_gkup_kernel = pl.pallas_call(
    _gkup_body,
    out_shape=[
        jax.ShapeDtypeStruct((G, H), f32),
        jax.ShapeDtypeStruct((G, H), f32),
        jax.ShapeDtypeStruct((G, H), f32),
    ],
)


def _gkfin_body(g_ref, nsum_ref, cnt_ref, gacc0_ref, gacc1_ref, Wg2_ref,
                bglob_ref, out_ref):
    gnew = _global_update(g_ref[...], nsum_ref[...], cnt_ref[...],
                          gacc0_ref[...], gacc1_ref[...], Wg2_ref[...],
                          bglob_ref[...])
    avg = nsum_ref[...] / jnp.clip(cnt_ref[...], 1.0, None)
    out_ref[...] = (jnp.sum(gnew, axis=1, keepdims=True)
                    + jnp.sum(avg, axis=1, keepdims=True))


_gkfin_kernel = pl.pallas_call(
    _gkfin_body,
    out_shape=[jax.ShapeDtypeStruct((G, 1), f32)],
)


def kernel(x, edge_index, edge_attr, global_attr, batch,
           Wx, bx, We, be, Wu, bu, Wedge, bedge, Wnode, bnode, Wglob, bglob):
    x = x.reshape(N, FN)
    edge_attr = edge_attr.reshape(E, FE)
    global_attr = global_attr.reshape(G, FG)
    src = edge_index[0].astype(jnp.int32)
    dst = edge_index[1].astype(jnp.int32)
    batch_i = batch.astype(jnp.int32)

    Ws = Wedge[:, 0:H, :]
    Wd = Wedge[:, H:2 * H, :]
    Wecol = Wedge[:, 2 * H:3 * H, :]
    Wgcol = Wedge[:, 3 * H:, :]
    Wn2 = Wnode[:, 0:2 * H, :]
    Wng = Wnode[:, 2 * H:, :]
    bx2 = bx.reshape(1, H)
    be2 = be.reshape(1, H)
    bu2 = bu.reshape(1, H)
    bedge2 = bedge.reshape(3, 1, H)
    bnode2 = bnode.reshape(3, 1, H)
    bglob2 = bglob.reshape(3, 1, H)

    b2 = batch_i.reshape(N, 1)

    g, gvec, gn, off_lo, off_hi, off_i = _gkenc_kernel(
        global_attr, b2, Wu, bu2, Wgcol[0], bedge2[0], Wng[0], bnode2[0])
    h, ps, pd = _u1_kernel(x, Wx, bx2, Ws[0], Wd[0], gvec, off_lo, off_hi)
    eW = _u2enc_kernel(edge_attr, We, be2, Wecol[0])[0]
    eg = _eg_kernel(src, off_i.reshape(G))

    e_new = None
    for t in range(3):
        if t > 0:
            eW = _u2mid_kernel(e_new, Wecol[t])[0]
            ps, pd = _uP_kernel(h, Ws[t], Wd[t], gvec, off_lo, off_hi)
        if t < 2:
            e_new, nacc, gacc = _edge_kernel_we(eW, ps, pd, src, dst, eg)
        else:
            nacc, gacc = _edge_kernel_last(eW, ps, pd, src, dst, eg)
        h, nsum, cnt = _u4_kernel(
            nacc[0], nacc[1], h, gn, Wn2[t], off_lo, off_hi)
        if t < 2:
            g, gvec, gn = _gkup_kernel(
                g, nsum, cnt, gacc[0], gacc[1], Wglob[t], bglob2[t],
                Wgcol[t + 1], bedge2[t + 1], Wng[t + 1], bnode2[t + 1])
        else:
            out = _gkfin_kernel(g, nsum, cnt, gacc[0], gacc[1], Wglob[t],
                                bglob2[t])[0]

    return out.reshape(G)
